# Initial kernel scaffold; baseline (speedup 1.0000x reference)
import jax
import jax.numpy as jnp
from jax.experimental import pallas as pl


def kernel(x, position_ids, cos_cached, sin_cached):
    raise NotImplementedError("write your pallas kernel here")


# speedup vs baseline: 1.4259x; 1.4259x over previous
import functools

import jax
import jax.numpy as jnp
from jax import lax
from jax.experimental import pallas as pl
from jax.experimental.pallas import tpu as pltpu
from jax.experimental.pallas import tpu_sc as plsc

_SEQ = 4096
_HEAD = 256


@jax.jit
def _rope_gather(cos_tab, sin_tab, idx):
    info = plsc.get_sparse_core_info()
    nw = info.num_cores * info.num_subcores
    b_per_w = _SEQ // nw
    mesh = plsc.VectorSubcoreMesh(core_axis_name="c", subcore_axis_name="s")

    @functools.partial(
        pl.kernel,
        mesh=mesh,
        out_type=[
            jax.ShapeDtypeStruct((_SEQ, _HEAD), jnp.float32),
            jax.ShapeDtypeStruct((_SEQ, _HEAD), jnp.float32),
        ],
        scratch_types=[
            pltpu.VMEM((b_per_w,), jnp.int32),
            pltpu.VMEM((b_per_w, _HEAD), jnp.float32),
            pltpu.VMEM((b_per_w, _HEAD), jnp.float32),
            pltpu.SemaphoreType.DMA,
            pltpu.SemaphoreType.DMA,
        ],
    )
    def k(cos_hbm, sin_hbm, idx_hbm, cos_out, sin_out, idx_v, cos_v, sin_v,
          sem_c, sem_s):
        wid = lax.axis_index("s") * info.num_cores + lax.axis_index("c")
        base = wid * b_per_w
        pltpu.sync_copy(idx_hbm.at[pl.ds(base, b_per_w)], idx_v)
        cpy_c = pltpu.async_copy(cos_hbm.at[idx_v], cos_v, sem_c)
        cpy_s = pltpu.async_copy(sin_hbm.at[idx_v], sin_v, sem_s)
        cpy_c.wait()
        pltpu.sync_copy(cos_v, cos_out.at[pl.ds(base, b_per_w)])
        cpy_s.wait()
        pltpu.sync_copy(sin_v, sin_out.at[pl.ds(base, b_per_w)])

    return k(cos_tab, sin_tab, idx)


def kernel(x, position_ids, cos_cached, sin_cached):
    idx = position_ids[0].astype(jnp.int32)
    cos, sin = _rope_gather(cos_cached[0], sin_cached[0], idx)
    return cos[None].astype(x.dtype), sin[None].astype(x.dtype)
